# baseline (speedup 1.0000x reference)
import jax
import jax.numpy as jnp
from jax.experimental import pallas as pl
from jax.experimental.pallas import tpu as pltpu

_K = 8


def _gcn_body(data_ref, adj_hbm, w_ref, b_ref, out_ref,
              buf_ref, mask_ref, sem):
    n = adj_hbm.shape[0]
    f = w_ref.shape[0]
    bn = n // _K

    def _copy(i, slot):
        return pltpu.make_async_copy(
            adj_hbm.at[pl.ds(i * bn, bn), :], buf_ref.at[slot], sem.at[slot])

    _copy(0, 0).start()
    _copy(1, 1).start()

    x = data_ref[...].reshape(2 * n, f)
    xw = jax.lax.dot_general(
        x, w_ref[...], (((1,), (1,)), ((), ())),
        preferred_element_type=jnp.float32)
    xw0 = xw[:n]
    xw1 = xw[n:]

    ones_col = jnp.ones((bn, 1), jnp.bfloat16)

    def _step(i, cnt):
        slot = jax.lax.rem(i, 2)
        _copy(i, slot).wait()
        mb = (buf_ref[slot] != 0.0).astype(jnp.bfloat16)
        mask_ref[pl.ds(i * bn, bn), :] = mb

        @pl.when(i + 2 < _K)
        def _prefetch():
            _copy(i + 2, slot).start()

        return cnt + jax.lax.dot_general(
            mb, ones_col, (((0,), (0,)), ((), ())),
            preferred_element_type=jnp.float32)

    cnt = jax.lax.fori_loop(0, _K, _step, jnp.zeros((n, 1), jnp.float32))

    nnz = jnp.sum(cnt)
    pad = jnp.float32(n) * jnp.float32(n) - nnz
    row_ids = jax.lax.broadcasted_iota(jnp.int32, (n, 1), 0)
    is_row0 = (row_ids == 0).astype(jnp.float32)
    deg = 2.0 * (cnt + pad * is_row0) + 1.0
    dis = jax.lax.rsqrt(deg)

    v = dis * xw0
    v_hi = v.astype(jnp.bfloat16)
    v_lo = (v - v_hi.astype(jnp.float32)).astype(jnp.bfloat16)
    mask = mask_ref[...]
    dims = (((0,), (0,)), ((), ()))
    s = (jax.lax.dot_general(mask, v_hi, dims,
                             preferred_element_type=jnp.float32)
         + jax.lax.dot_general(mask, v_lo, dims,
                               preferred_element_type=jnp.float32))
    s = s + is_row0 * (pad * v[0:1, :])

    b_row = b_ref[...]
    out_ref[0] = (2.0 * dis) * s + (dis * dis) * xw0 + b_row
    out_ref[1] = xw1 + b_row


def kernel(data, adj, W, b):
    batch, n, f = data.shape
    bn = n // _K
    return pl.pallas_call(
        _gcn_body,
        in_specs=[
            pl.BlockSpec(memory_space=pltpu.MemorySpace.VMEM),
            pl.BlockSpec(memory_space=pl.ANY),
            pl.BlockSpec(memory_space=pltpu.MemorySpace.VMEM),
            pl.BlockSpec(memory_space=pltpu.MemorySpace.VMEM),
        ],
        out_specs=pl.BlockSpec(memory_space=pltpu.MemorySpace.VMEM),
        out_shape=jax.ShapeDtypeStruct((batch, n, f), data.dtype),
        scratch_shapes=[
            pltpu.VMEM((2, bn, n), jnp.float32),
            pltpu.VMEM((n, n), jnp.bfloat16),
            pltpu.SemaphoreType.DMA((2,)),
        ],
    )(data, adj, W, b.reshape(1, f))


# speedup vs baseline: 1.2504x; 1.2504x over previous
import jax
import jax.numpy as jnp
from jax.experimental import pallas as pl


def _gcn_body(data_ref, adj_ref, w_ref, b_ref, out_ref):
    n = adj_ref.shape[0]
    f = w_ref.shape[0]
    adj = adj_ref[...]
    mask = (adj != 0.0).astype(jnp.bfloat16)

    ones_col = jnp.ones((n, 1), jnp.bfloat16)
    cnt = jax.lax.dot_general(
        mask, ones_col, (((0,), (0,)), ((), ())),
        preferred_element_type=jnp.float32)
    nnz = jnp.sum(cnt)
    pad = jnp.float32(n) * jnp.float32(n) - nnz

    row_ids = jax.lax.broadcasted_iota(jnp.int32, (n, 1), 0)
    is_row0 = (row_ids == 0).astype(jnp.float32)
    cnt = cnt + pad * is_row0
    deg = 2.0 * cnt + 1.0
    dis = jax.lax.rsqrt(deg)

    x = data_ref[...].reshape(2 * n, f)
    xw = jax.lax.dot_general(
        x, w_ref[...], (((1,), (1,)), ((), ())),
        preferred_element_type=jnp.float32)
    xw0 = xw[:n]
    xw1 = xw[n:]

    v = dis * xw0
    v_hi = v.astype(jnp.bfloat16)
    dims = (((0,), (0,)), ((), ()))
    s = jax.lax.dot_general(mask, v_hi, dims,
                            preferred_element_type=jnp.float32)
    s = s + is_row0 * (pad * v[0:1, :])

    b_row = b_ref[...]
    out_ref[0] = (2.0 * dis) * s + (dis * dis) * xw0 + b_row
    out_ref[1] = xw1 + b_row


def kernel(data, adj, W, b):
    batch, n, f = data.shape
    return pl.pallas_call(
        _gcn_body,
        out_shape=jax.ShapeDtypeStruct((batch, n, f), data.dtype),
    )(data, adj, W, b.reshape(1, f))
